# Initial kernel scaffold; baseline (speedup 1.0000x reference)
import jax
import jax.numpy as jnp
from jax.experimental import pallas as pl


def kernel(input_ids, witness_ids, witness_weight):
    raise NotImplementedError("write your pallas kernel here")


# speedup vs baseline: 97.5288x; 97.5288x over previous
import functools

import jax
import jax.numpy as jnp
from jax import lax
from jax.experimental import pallas as pl
from jax.experimental.pallas import tpu as pltpu
from jax.experimental.pallas import tpu_sc as plsc

_NC, _NS, _LANES = 2, 16, 16
_NW = _NC * _NS

_IDX_W = 128


@functools.lru_cache(maxsize=None)
def _make_gather(n_rows: int, rows_per_w: int, k_rows: int):
    n_outer = rows_per_w // k_rows
    mesh = plsc.VectorSubcoreMesh(core_axis_name="c", subcore_axis_name="s")

    @functools.partial(
        pl.kernel,
        out_type=jax.ShapeDtypeStruct((n_rows, _IDX_W), jnp.float32),
        mesh=mesh,
        scratch_types=[
            pltpu.VMEM((k_rows, _IDX_W), jnp.int32),
            pltpu.VMEM((k_rows, _IDX_W), jnp.float32),
            pltpu.SemaphoreType.DMA,
        ],
    )
    def gather_kernel(table_hbm, ids_hbm, out_hbm, idx_v, vals_v, sem):
        wid = lax.axis_index("s") * _NC + lax.axis_index("c")
        w_base = wid * rows_per_w

        def body(g, carry):
            row_base = w_base + g * k_rows
            pltpu.sync_copy(ids_hbm.at[pl.ds(row_base, k_rows)], idx_v)
            descs = [
                pltpu.async_copy(table_hbm.at[idx_v.at[j]], vals_v.at[j], sem)
                for j in range(k_rows)
            ]
            for d in descs:
                d.wait()
            for j in range(k_rows):
                for i in range(_IDX_W // _LANES):
                    v = vals_v[j, pl.ds(i * _LANES, _LANES)]
                    vals_v[j, pl.ds(i * _LANES, _LANES)] = v - v
            pltpu.sync_copy(vals_v, out_hbm.at[pl.ds(row_base, k_rows)])
            return carry

        lax.fori_loop(0, n_outer, body, 0)

    return gather_kernel


def kernel(input_ids, witness_ids, witness_weight):
    b, l = witness_ids.shape
    v = witness_weight.shape[0]
    n = b * l
    assert n % (_NW * _IDX_W) == 0
    n_rows = n // _IDX_W
    rows_per_w = n_rows // _NW
    k_rows = 16 if rows_per_w % 16 == 0 else 8
    assert rows_per_w % k_rows == 0

    ids2d = witness_ids.reshape(n_rows, _IDX_W)
    table = witness_weight.reshape(v)
    out2d = _make_gather(n_rows, rows_per_w, k_rows)(table, ids2d)
    return out2d.reshape(b, l, 1)


# speedup vs baseline: 149.6636x; 1.5346x over previous
import functools

import jax
import jax.numpy as jnp
from jax import lax
from jax.experimental import pallas as pl
from jax.experimental.pallas import tpu as pltpu
from jax.experimental.pallas import tpu_sc as plsc

_NC, _NS, _LANES = 2, 16, 16
_NW = _NC * _NS

_IDX_W = 128


@functools.lru_cache(maxsize=None)
def _make_gather(n_rows: int, rows_per_w: int, k_rows: int, v: int):
    n_outer = rows_per_w // k_rows
    mesh = plsc.VectorSubcoreMesh(core_axis_name="c", subcore_axis_name="s")

    @functools.partial(
        pl.kernel,
        out_type=jax.ShapeDtypeStruct((n_rows, _IDX_W), jnp.float32),
        mesh=mesh,
        scratch_types=[
            pltpu.VMEM((k_rows, _IDX_W), jnp.int32),
            pltpu.VMEM((k_rows, _IDX_W), jnp.float32),
            pltpu.VMEM_SHARED((v,), jnp.float32),
            pltpu.SemaphoreType.DMA,
        ],
    )
    def gather_kernel(table_hbm, ids_hbm, out_hbm, idx_v, vals_v, tab_s, sem):
        s = lax.axis_index("s")
        wid = s * _NC + lax.axis_index("c")
        w_base = wid * rows_per_w

        @pl.when(s == 0)
        def _stage():
            pltpu.sync_copy(table_hbm, tab_s)

        plsc.subcore_barrier()

        def body(g, carry):
            row_base = w_base + g * k_rows
            pltpu.sync_copy(ids_hbm.at[pl.ds(row_base, k_rows)], idx_v)
            descs = [
                pltpu.async_copy(tab_s.at[idx_v.at[j]], vals_v.at[j], sem)
                for j in range(k_rows)
            ]
            for d in descs:
                d.wait()
            for j in range(k_rows):
                for i in range(_IDX_W // _LANES):
                    v = vals_v[j, pl.ds(i * _LANES, _LANES)]
                    vals_v[j, pl.ds(i * _LANES, _LANES)] = v - v
            pltpu.sync_copy(vals_v, out_hbm.at[pl.ds(row_base, k_rows)])
            return carry

        lax.fori_loop(0, n_outer, body, 0)

    return gather_kernel


def kernel(input_ids, witness_ids, witness_weight):
    b, l = witness_ids.shape
    v = witness_weight.shape[0]
    n = b * l
    assert n % (_NW * _IDX_W) == 0
    n_rows = n // _IDX_W
    rows_per_w = n_rows // _NW
    k_rows = 16 if rows_per_w % 16 == 0 else 8
    assert rows_per_w % k_rows == 0

    ids2d = witness_ids.reshape(n_rows, _IDX_W)
    table = witness_weight.reshape(v)
    out2d = _make_gather(n_rows, rows_per_w, k_rows, v)(table, ids2d)
    return out2d.reshape(b, l, 1)


# speedup vs baseline: 165.6863x; 1.1071x over previous
import functools

import jax
import jax.numpy as jnp
from jax import lax
from jax.experimental import pallas as pl
from jax.experimental.pallas import tpu as pltpu
from jax.experimental.pallas import tpu_sc as plsc

_NC, _NS, _LANES = 2, 16, 16
_NW = _NC * _NS


@functools.lru_cache(maxsize=None)
def _make_gather(b: int, l: int, v: int, kb: int):
    rows_per_w = b // _NW
    n_outer = rows_per_w // kb
    segs = []
    off = 0
    while off < l:
        segs.append((off, min(128, l - off)))
        off += min(128, l - off)

    mesh = plsc.VectorSubcoreMesh(core_axis_name="c", subcore_axis_name="s")

    @functools.partial(
        pl.kernel,
        out_type=jax.ShapeDtypeStruct((b, l), jnp.float32),
        mesh=mesh,
        scratch_types=[
            pltpu.VMEM((kb, l), jnp.int32),
            pltpu.VMEM((kb, l), jnp.float32),
            pltpu.VMEM_SHARED((v,), jnp.float32),
            pltpu.SemaphoreType.DMA,
        ],
    )
    def gather_kernel(table_hbm, ids_hbm, out_hbm, idx_v, vals_v, tab_s, sem):
        s = lax.axis_index("s")
        wid = s * _NC + lax.axis_index("c")
        w_base = wid * rows_per_w

        @pl.when(s == 0)
        def _stage():
            pltpu.sync_copy(table_hbm, tab_s)

        plsc.subcore_barrier()

        def body(g, carry):
            row_base = w_base + g * kb
            pltpu.sync_copy(ids_hbm.at[pl.ds(row_base, kb)], idx_v)
            descs = [
                pltpu.async_copy(
                    tab_s.at[idx_v.at[j, pl.ds(o, w)]],
                    vals_v.at[j, pl.ds(o, w)],
                    sem,
                )
                for j in range(kb)
                for (o, w) in segs
            ]
            for d in descs:
                d.wait()
            starts = [i * _LANES for i in range(l // _LANES)]
            if l % _LANES:
                starts.append(l - _LANES)
            for j in range(kb):
                for st in starts:
                    val = vals_v[j, pl.ds(st, _LANES)]
                    vals_v[j, pl.ds(st, _LANES)] = val - val
            pltpu.sync_copy(vals_v, out_hbm.at[pl.ds(row_base, kb)])
            return carry

        lax.fori_loop(0, n_outer, body, 0)

    return gather_kernel


def kernel(input_ids, witness_ids, witness_weight):
    b, l = witness_ids.shape
    v = witness_weight.shape[0]
    assert b % (_NW * 8) == 0
    kb = 8
    table = witness_weight.reshape(v)
    out2d = _make_gather(b, l, v, kb)(table, witness_ids)
    return out2d.reshape(b, l, 1)


# speedup vs baseline: 202.8208x; 1.2241x over previous
import functools

import jax
import jax.numpy as jnp
from jax import lax
from jax.experimental import pallas as pl
from jax.experimental.pallas import tpu as pltpu
from jax.experimental.pallas import tpu_sc as plsc

_NC, _NS, _LANES = 2, 16, 16
_NW = _NC * _NS


@functools.lru_cache(maxsize=None)
def _make_gather(b: int, l: int, v: int, kl: int):
    cols_per_w = b // _NW
    n_outer = l // kl
    segs = [(o, 128) for o in range(0, cols_per_w, 128)]

    mesh = plsc.VectorSubcoreMesh(core_axis_name="c", subcore_axis_name="s")

    @functools.partial(
        pl.kernel,
        out_type=jax.ShapeDtypeStruct((b * l,), jnp.float32),
        mesh=mesh,
        scratch_types=[
            pltpu.VMEM((kl, cols_per_w), jnp.int32),
            pltpu.VMEM((kl, cols_per_w), jnp.float32),
            pltpu.VMEM_SHARED((v,), jnp.float32),
            pltpu.SemaphoreType.DMA,
        ],
    )
    def gather_kernel(table_hbm, idst_hbm, out_hbm, idx_v, vals_v, tab_s, sem):
        s = lax.axis_index("s")
        wid = s * _NC + lax.axis_index("c")
        wb = wid * cols_per_w

        @pl.when(s == 0)
        def _stage():
            pltpu.sync_copy(table_hbm, tab_s)

        plsc.subcore_barrier()

        def body(g, carry):
            l0 = g * kl
            pltpu.sync_copy(idst_hbm.at[pl.ds(l0, kl), pl.ds(wb, cols_per_w)], idx_v)
            descs = [
                pltpu.async_copy(
                    tab_s.at[idx_v.at[j, pl.ds(o, w)]],
                    vals_v.at[j, pl.ds(o, w)],
                    sem,
                )
                for j in range(kl)
                for (o, w) in segs
            ]
            for d in descs:
                d.wait()
            for j in range(kl):
                for i in range(cols_per_w // _LANES):
                    val = vals_v[j, pl.ds(i * _LANES, _LANES)]
                    vals_v[j, pl.ds(i * _LANES, _LANES)] = val - val
            wdescs = [
                pltpu.async_copy(
                    vals_v.at[j],
                    out_hbm.at[pl.ds((l0 + j) * b + wb, cols_per_w)],
                    sem,
                )
                for j in range(kl)
            ]
            for d in wdescs:
                d.wait()
            return carry

        lax.fori_loop(0, n_outer, body, 0)

    return gather_kernel


def kernel(input_ids, witness_ids, witness_weight):
    b, l = witness_ids.shape
    v = witness_weight.shape[0]
    kl = 8
    table = jnp.squeeze(witness_weight, axis=1)
    ids_t = witness_ids.T
    out1d = _make_gather(b, l, v, kl)(table, ids_t)
    return out1d.reshape(l, b).T.reshape(b, l, 1)


# speedup vs baseline: 245.3753x; 1.2098x over previous
import functools

import jax
import jax.numpy as jnp
from jax import lax
from jax.experimental import pallas as pl
from jax.experimental.pallas import tpu as pltpu
from jax.experimental.pallas import tpu_sc as plsc

_NC, _NS, _LANES = 2, 16, 16
_NW = _NC * _NS


@functools.lru_cache(maxsize=None)
def _make_gather(b: int, l: int, v: int, kl: int):
    cols_per_w = b // _NW
    n_outer = l // kl
    segs = [(o, 128) for o in range(0, cols_per_w, 128)]

    mesh = plsc.VectorSubcoreMesh(core_axis_name="c", subcore_axis_name="s")

    @functools.partial(
        pl.kernel,
        out_type=jax.ShapeDtypeStruct((b * l,), jnp.float32),
        mesh=mesh,
        scratch_types=[
            pltpu.VMEM((kl, cols_per_w), jnp.int32),
            pltpu.VMEM((kl, cols_per_w), jnp.float32),
            pltpu.VMEM_SHARED((v,), jnp.float32),
            pltpu.SemaphoreType.DMA,
        ],
    )
    def gather_kernel(table_hbm, idst_hbm, out_hbm, idx_v, vals_v, tab_s, sem):
        s = lax.axis_index("s")
        wid = s * _NC + lax.axis_index("c")
        wb = wid * cols_per_w

        @pl.when(s == 0)
        def _stage():
            pltpu.sync_copy(table_hbm, tab_s)

        plsc.subcore_barrier()

        def body(g, carry):
            l0 = g * kl
            pltpu.sync_copy(idst_hbm.at[pl.ds(l0, kl), pl.ds(wb, cols_per_w)], idx_v)
            descs = [
                pltpu.async_copy(
                    tab_s.at[idx_v.at[j, pl.ds(o, w)]],
                    vals_v.at[j, pl.ds(o, w)],
                    sem,
                )
                for j in range(kl)
                for (o, w) in segs
            ]
            for d in descs:
                d.wait()
            for j in range(kl):
                for i in range(cols_per_w // _LANES):
                    val = vals_v[j, pl.ds(i * _LANES, _LANES)]
                    vals_v[j, pl.ds(i * _LANES, _LANES)] = val - val
            wdescs = [
                pltpu.async_copy(
                    vals_v.at[j],
                    out_hbm.at[pl.ds((l0 + j) * b + wb, cols_per_w)],
                    sem,
                )
                for j in range(kl)
            ]
            for d in wdescs:
                d.wait()
            return carry

        lax.fori_loop(0, n_outer, body, 0)

    return gather_kernel


def kernel(input_ids, witness_ids, witness_weight):
    b, l = witness_ids.shape
    v = witness_weight.shape[0]
    kl = 8
    table = witness_weight.T.reshape(v)
    ids_t = witness_ids.T
    out1d = _make_gather(b, l, v, kl)(table, ids_t)
    return out1d.reshape(l, b, 1).transpose(1, 0, 2)


# speedup vs baseline: 300.4884x; 1.2246x over previous
import functools

import jax
import jax.numpy as jnp
from jax import lax
from jax.experimental import pallas as pl
from jax.experimental.pallas import tpu as pltpu
from jax.experimental.pallas import tpu_sc as plsc

_NC, _NS, _LANES = 2, 16, 16
_NW = _NC * _NS


@functools.lru_cache(maxsize=None)
def _make_gather(b: int, l: int, v: int, kl: int):
    cols_per_w = b // _NW
    n_outer = l // kl
    assert n_outer % 2 == 1 and n_outer >= 3
    half_iters = (n_outer - 1) // 2
    segs = [(o, 128) for o in range(0, cols_per_w, 128)]

    mesh = plsc.VectorSubcoreMesh(core_axis_name="c", subcore_axis_name="s")

    @functools.partial(
        pl.kernel,
        out_type=jax.ShapeDtypeStruct((b * l,), jnp.float32),
        mesh=mesh,
        scratch_types=[
            pltpu.VMEM((kl, cols_per_w), jnp.int32),
            pltpu.VMEM((kl, cols_per_w), jnp.int32),
            pltpu.VMEM((kl, cols_per_w), jnp.float32),
            pltpu.VMEM((kl, cols_per_w), jnp.float32),
            pltpu.VMEM_SHARED((v,), jnp.float32),
            pltpu.SemaphoreType.DMA,
            pltpu.SemaphoreType.DMA,
            pltpu.SemaphoreType.DMA,
            pltpu.SemaphoreType.DMA,
            pltpu.SemaphoreType.DMA,
        ],
    )
    def gather_kernel(table_hbm, idst_hbm, out_hbm,
                      idx_a, idx_b, vals_a, vals_b, tab_s,
                      sia, sib, sg, swa, swb):
        s = lax.axis_index("s")
        wid = s * _NC + lax.axis_index("c")
        wb = wid * cols_per_w

        @pl.when(s == 0)
        def _stage():
            pltpu.sync_copy(table_hbm, tab_s)

        plsc.subcore_barrier()

        def ids_src(l0):
            return idst_hbm.at[pl.ds(l0, kl), pl.ds(wb, cols_per_w)]

        def issue_idx(l0, idx_v, sem):
            pltpu.async_copy(ids_src(l0), idx_v, sem)

        def wait_idx(idx_v, sem):
            pltpu.make_async_copy(ids_src(0), idx_v, sem).wait()

        def wait_writes(idx_v, sem):
            pltpu.make_async_copy(ids_src(0), idx_v, sem).wait()

        def process(g_dyn, idx_v, vals_v, sw):
            l0 = g_dyn * kl
            descs = [
                pltpu.async_copy(
                    tab_s.at[idx_v.at[j, pl.ds(o, w)]],
                    vals_v.at[j, pl.ds(o, w)],
                    sg,
                )
                for j in range(kl)
                for (o, w) in segs
            ]
            for d in descs:
                d.wait()
            for j in range(kl):
                for i in range(cols_per_w // _LANES):
                    val = vals_v[j, pl.ds(i * _LANES, _LANES)]
                    vals_v[j, pl.ds(i * _LANES, _LANES)] = val - val
            for j in range(kl):
                pltpu.async_copy(
                    vals_v.at[j],
                    out_hbm.at[pl.ds((l0 + j) * b + wb, cols_per_w)],
                    sw,
                )

        issue_idx(0, idx_a, sia)
        wait_idx(idx_a, sia)
        issue_idx(kl, idx_b, sib)
        process(0, idx_a, vals_a, swa)

        def body(t, carry):
            wait_idx(idx_b, sib)
            issue_idx((2 * t + 2) * kl, idx_a, sia)

            @pl.when(t > 0)
            def _():
                wait_writes(idx_b, swb)

            process(2 * t + 1, idx_b, vals_b, swb)

            wait_idx(idx_a, sia)

            @pl.when(t < half_iters - 1)
            def _():
                issue_idx((2 * t + 3) * kl, idx_b, sib)

            wait_writes(idx_a, swa)
            process(2 * t + 2, idx_a, vals_a, swa)
            return carry

        lax.fori_loop(0, half_iters, body, 0)

        wait_writes(idx_a, swa)
        wait_writes(idx_b, swb)

    return gather_kernel


def kernel(input_ids, witness_ids, witness_weight):
    b, l = witness_ids.shape
    v = witness_weight.shape[0]
    kl = 8
    table = jnp.squeeze(witness_weight, axis=1)
    ids_t = witness_ids.T
    out1d = _make_gather(b, l, v, kl)(table, ids_t)
    return out1d.reshape(l, b, 1).transpose(1, 0, 2)
